# speedup vs baseline: 6.3331x; 6.3331x over previous
import jax
import jax.numpy as jnp
from jax import lax
from jax.experimental import pallas as pl
from jax.experimental.pallas import tpu as pltpu
from jax.experimental.pallas import tpu_sc as plsc

NUM_EMBEDDINGS = 100000
EMBEDDING_DIM = 128
BATCH = 4096
SEQ = 200

_NC = 2
_NS = 16
_NW = _NC * _NS

_B = BATCH * SEQ
_G = 128
_ROWS_PER_W = _B // _NW
_STEPS = _ROWS_PER_W // _G


def _body(idx_hbm, table_hbm, out_hbm, idx_v, rows_v, gsem):
    wid = lax.axis_index("s") * _NC + lax.axis_index("c")
    pltpu.sync_copy(idx_hbm.at[pl.ds(wid * _STEPS, _STEPS)], idx_v)
    row_base = wid * _ROWS_PER_W

    def step(s, carry):
        pltpu.async_copy(table_hbm.at[idx_v.at[s]], rows_v, gsem).wait()
        pltpu.sync_copy(rows_v, out_hbm.at[pl.ds(row_base + s * _G, _G)])
        return carry

    lax.fori_loop(0, _STEPS, step, 0)


@jax.jit
def kernel(x, param):
    idx = x.reshape(_B // _G, _G).astype(jnp.int32)
    mesh = plsc.VectorSubcoreMesh(core_axis_name="c", subcore_axis_name="s")
    out = pl.kernel(
        _body,
        out_type=jax.ShapeDtypeStruct((_B, EMBEDDING_DIM), jnp.float32),
        mesh=mesh,
        scratch_types=[
            pltpu.VMEM((_STEPS, _G), jnp.int32),
            pltpu.VMEM((_G, EMBEDDING_DIM), jnp.float32),
            pltpu.SemaphoreType.DMA,
        ],
    )(idx, param)
    return out.reshape(BATCH, SEQ, EMBEDDING_DIM)


# speedup vs baseline: 9.1180x; 1.4397x over previous
import jax
import jax.numpy as jnp
from jax import lax
from jax.experimental import pallas as pl
from jax.experimental.pallas import tpu as pltpu
from jax.experimental.pallas import tpu_sc as plsc

NUM_EMBEDDINGS = 100000
EMBEDDING_DIM = 128
BATCH = 4096
SEQ = 200

_NC = 2
_NS = 16
_NW = _NC * _NS

_B = BATCH * SEQ
_G = 128
_ROWS_PER_W = _B // _NW
_STEPS = _ROWS_PER_W // _G
_NBUF = 4
_T = _STEPS // _NBUF


def _body(idx_hbm, table_hbm, out_hbm, ibufs, rbufs, gsem, osem):
    wid = lax.axis_index("s") * _NC + lax.axis_index("c")
    idx_row0 = wid * _STEPS
    row_base = wid * _ROWS_PER_W

    def load_idx(s, j):
        pltpu.sync_copy(idx_hbm.at[idx_row0 + s], ibufs.at[j])

    def start_gather(j):
        pltpu.async_copy(table_hbm.at[ibufs.at[j]], rbufs.at[j], gsem)

    def wait_gather(j):
        pltpu.make_async_copy(table_hbm.at[ibufs.at[j]], rbufs.at[j], gsem).wait()

    def start_scatter(s, j):
        pltpu.async_copy(rbufs.at[j], out_hbm.at[pl.ds(row_base + s * _G, _G)], osem)

    def wait_scatter(j):
        pltpu.make_async_copy(
            rbufs.at[j], out_hbm.at[pl.ds(row_base, _G)], osem
        ).wait()

    load_idx(0, 0)
    start_gather(0)
    load_idx(1, 1)
    start_gather(1)

    for j in range(_NBUF):
        wait_gather(j)
        start_scatter(j, j)
        nj = (j + 2) % _NBUF
        load_idx(j + 2, nj)
        if j >= 2:
            wait_scatter(nj)
        start_gather(nj)

    def pass_body(t, carry):
        s0 = t * _NBUF
        for j in range(_NBUF):
            wait_gather(j)
            start_scatter(s0 + j, j)
            nj = (j + 2) % _NBUF
            load_idx(s0 + j + 2, nj)
            wait_scatter(nj)
            start_gather(nj)
        return carry

    lax.fori_loop(1, _T - 1, pass_body, 0)

    s0 = _STEPS - _NBUF
    for j in range(_NBUF):
        wait_gather(j)
        start_scatter(s0 + j, j)
        if j < 2:
            nj = (j + 2) % _NBUF
            load_idx(s0 + j + 2, nj)
            wait_scatter(nj)
            start_gather(nj)

    for _ in range(_NBUF):
        wait_scatter(0)


@jax.jit
def kernel(x, param):
    idx = x.reshape(_B // _G, _G).astype(jnp.int32)
    mesh = plsc.VectorSubcoreMesh(core_axis_name="c", subcore_axis_name="s")
    out = pl.kernel(
        _body,
        out_type=jax.ShapeDtypeStruct((_B, EMBEDDING_DIM), jnp.float32),
        mesh=mesh,
        scratch_types=[
            pltpu.VMEM((_NBUF, _G), jnp.int32),
            pltpu.VMEM((_NBUF, _G, EMBEDDING_DIM), jnp.float32),
            pltpu.SemaphoreType.DMA,
            pltpu.SemaphoreType.DMA,
        ],
    )(idx, param)
    return out.reshape(BATCH, SEQ, EMBEDDING_DIM)


# speedup vs baseline: 9.2464x; 1.0141x over previous
import jax
import jax.numpy as jnp
from jax import lax
from jax.experimental import pallas as pl
from jax.experimental.pallas import tpu as pltpu
from jax.experimental.pallas import tpu_sc as plsc

NUM_EMBEDDINGS = 100000
EMBEDDING_DIM = 128
BATCH = 4096
SEQ = 200

_NC = 2
_NS = 16
_NW = _NC * _NS

_B = BATCH * SEQ
_G = 128
_ROWS_PER_W = _B // _NW
_STEPS = _ROWS_PER_W // _G
_NRB = 4
_NIB = 8
_T = _STEPS // _NIB


def _body(idx_hbm, table_hbm, out_hbm, ibufs, rbufs, gsem, osem, isem):
    wid = lax.axis_index("s") * _NC + lax.axis_index("c")
    idx_row0 = wid * _STEPS
    row_base = wid * _ROWS_PER_W

    def start_idx(s, k):
        pltpu.async_copy(idx_hbm.at[idx_row0 + s], ibufs.at[k], isem)

    def wait_idx(k):
        pltpu.make_async_copy(idx_hbm.at[idx_row0], ibufs.at[k], isem).wait()

    def start_gather(j, k):
        pltpu.async_copy(table_hbm.at[ibufs.at[k]], rbufs.at[j], gsem)

    def wait_gather(j, k):
        pltpu.make_async_copy(table_hbm.at[ibufs.at[k]], rbufs.at[j], gsem).wait()

    def start_scatter(s, j):
        pltpu.async_copy(rbufs.at[j], out_hbm.at[pl.ds(row_base + s * _G, _G)], osem)

    def wait_scatter(j):
        pltpu.make_async_copy(
            rbufs.at[j], out_hbm.at[pl.ds(row_base, _G)], osem
        ).wait()

    def step(s, j8, first_pass):
        j = j8 % _NRB
        wait_gather(j, j8)
        start_scatter(s, j)
        k2 = (j8 + 2) % _NIB
        wait_idx(k2)
        if not (first_pass and j8 < 2):
            wait_scatter((j + 2) % _NRB)
        start_gather((j + 2) % _NRB, k2)
        start_idx(s + 6, (j8 + 6) % _NIB)

    for k in range(6):
        start_idx(k, k)
    wait_idx(0)
    wait_idx(1)
    start_gather(0, 0)
    start_gather(1, 1)

    for j8 in range(_NIB):
        step(j8, j8, True)

    def pass_body(t, carry):
        s0 = t * _NIB
        for j8 in range(_NIB):
            step(s0 + j8, j8, False)
        return carry

    lax.fori_loop(1, _T - 1, pass_body, 0)

    s0 = _STEPS - _NIB
    for j8 in range(_NIB):
        s = s0 + j8
        j = j8 % _NRB
        wait_gather(j, j8)
        start_scatter(s, j)
        if j8 < _NIB - 2:
            k2 = (j8 + 2) % _NIB
            wait_idx(k2)
            wait_scatter((j + 2) % _NRB)
            start_gather((j + 2) % _NRB, k2)
        if j8 < 2:
            start_idx(s + 6, (j8 + 6) % _NIB)

    for _ in range(_NRB):
        wait_scatter(0)


@jax.jit
def kernel(x, param):
    idx = x.reshape(_B // _G, _G).astype(jnp.int32)
    mesh = plsc.VectorSubcoreMesh(core_axis_name="c", subcore_axis_name="s")
    out = pl.kernel(
        _body,
        out_type=jax.ShapeDtypeStruct((_B, EMBEDDING_DIM), jnp.float32),
        mesh=mesh,
        scratch_types=[
            pltpu.VMEM((_NIB, _G), jnp.int32),
            pltpu.VMEM((_NRB, _G, EMBEDDING_DIM), jnp.float32),
            pltpu.SemaphoreType.DMA,
            pltpu.SemaphoreType.DMA,
            pltpu.SemaphoreType.DMA,
        ],
    )(idx, param)
    return out.reshape(BATCH, SEQ, EMBEDDING_DIM)


# speedup vs baseline: 9.2669x; 1.0022x over previous
import jax
import jax.numpy as jnp
from jax import lax
from jax.experimental import pallas as pl
from jax.experimental.pallas import tpu as pltpu
from jax.experimental.pallas import tpu_sc as plsc

NUM_EMBEDDINGS = 100000
EMBEDDING_DIM = 128
BATCH = 4096
SEQ = 200

_NC = 2
_NS = 16
_NW = _NC * _NS

_B = BATCH * SEQ
_G = 128
_ROWS_PER_W = _B // _NW
_STEPS = _ROWS_PER_W // _G
_NRB = 5
_GP = 3
_NIB = 10
_T = _STEPS // _NIB


def _body(idx_hbm, table_hbm, out_hbm, ibufs, rbufs, gsem, osem, isem):
    wid = lax.axis_index("s") * _NC + lax.axis_index("c")
    idx_row0 = wid * _STEPS
    row_base = wid * _ROWS_PER_W

    def start_idx(s, k):
        pltpu.async_copy(idx_hbm.at[idx_row0 + s], ibufs.at[k], isem)

    def wait_idx(k):
        pltpu.make_async_copy(idx_hbm.at[idx_row0], ibufs.at[k], isem).wait()

    def start_gather(j, k):
        pltpu.async_copy(table_hbm.at[ibufs.at[k]], rbufs.at[j], gsem)

    def wait_gather(j, k):
        pltpu.make_async_copy(table_hbm.at[ibufs.at[k]], rbufs.at[j], gsem).wait()

    def start_scatter(s, j):
        pltpu.async_copy(rbufs.at[j], out_hbm.at[pl.ds(row_base + s * _G, _G)], osem)

    def wait_scatter(j):
        pltpu.make_async_copy(
            rbufs.at[j], out_hbm.at[pl.ds(row_base, _G)], osem
        ).wait()

    def step(s, jI, first_pass):
        j = jI % _NRB
        wait_gather(j, jI)
        start_scatter(s, j)
        kg = (jI + _GP) % _NIB
        wait_idx(kg)
        if not (first_pass and jI < 2):
            wait_scatter((j + _GP) % _NRB)
        start_gather((j + _GP) % _NRB, kg)
        start_idx(s + 6, (jI + 6) % _NIB)

    for k in range(6):
        start_idx(k, k)
    for k in range(_GP):
        wait_idx(k)
        start_gather(k, k)

    for jI in range(_NIB):
        step(jI, jI, True)

    def pass_body(t, carry):
        s0 = t * _NIB
        for jI in range(_NIB):
            step(s0 + jI, jI, False)
        return carry

    lax.fori_loop(1, _T - 1, pass_body, 0)

    s0 = _STEPS - _NIB
    for jI in range(_NIB):
        s = s0 + jI
        j = jI % _NRB
        wait_gather(j, jI)
        start_scatter(s, j)
        if jI < _NIB - _GP:
            kg = (jI + _GP) % _NIB
            wait_idx(kg)
            wait_scatter((j + _GP) % _NRB)
            start_gather((j + _GP) % _NRB, kg)
        if jI < 4:
            start_idx(s + 6, (jI + 6) % _NIB)

    for _ in range(_NRB):
        wait_scatter(0)


@jax.jit
def kernel(x, param):
    idx = x.reshape(_B // _G, _G).astype(jnp.int32)
    mesh = plsc.VectorSubcoreMesh(core_axis_name="c", subcore_axis_name="s")
    out = pl.kernel(
        _body,
        out_type=jax.ShapeDtypeStruct((_B, EMBEDDING_DIM), jnp.float32),
        mesh=mesh,
        scratch_types=[
            pltpu.VMEM((_NIB, _G), jnp.int32),
            pltpu.VMEM((_NRB, _G, EMBEDDING_DIM), jnp.float32),
            pltpu.SemaphoreType.DMA,
            pltpu.SemaphoreType.DMA,
            pltpu.SemaphoreType.DMA,
        ],
    )(idx, param)
    return out.reshape(BATCH, SEQ, EMBEDDING_DIM)
